# baseline (speedup 1.0000x reference)
import functools

import jax
import jax.numpy as jnp
import numpy as np
from jax import lax
from jax.experimental import pallas as pl
from jax.experimental.pallas import tpu as pltpu
from jax.experimental.pallas import tpu_sc as plsc

D = 16
EPS = 1e-05
MAX_NORM = 1.0 - EPS


def _sc_gather(table, idx2d):
    n_rows, G = idx2d.shape
    M = n_rows * G
    info = plsc.get_sparse_core_info()
    NC, NS = info.num_cores, info.num_subcores
    NW = NC * NS
    NBUF = 2
    K = 8
    CH = K * G
    per_w = M // NW
    n_chunks = per_w // CH
    n_outer = n_chunks // NBUF

    mesh = plsc.VectorSubcoreMesh(core_axis_name="c", subcore_axis_name="s")

    @functools.partial(
        pl.kernel,
        mesh=mesh,
        out_type=jax.ShapeDtypeStruct((M, D), jnp.float32),
        scratch_types=[
            pltpu.VMEM((NBUF, K, G), jnp.int32),
            pltpu.VMEM((NBUF, CH, D), jnp.float32),
            [pltpu.SemaphoreType.DMA] * NBUF,
            [pltpu.SemaphoreType.DMA] * NBUF,
            [pltpu.SemaphoreType.DMA] * NBUF,
        ],
        compiler_params=pltpu.CompilerParams(use_tc_tiling_on_sc=False),
    )
    def gather_kernel(table_hbm, idx_hbm, out_hbm, idx_v, rows_v,
                      isems, gsems, wsems):
        wid = lax.axis_index("s") * NC + lax.axis_index("c")
        base = wid * per_w

        def idx_row(c):
            return pl.multiple_of((base // G) + c * K, K)

        def start_idx(c, b):
            pltpu.async_copy(idx_hbm.at[pl.ds(idx_row(c), K)],
                             idx_v.at[b], isems[b])

        def start_gathers(b):
            for j in range(K):
                pltpu.async_copy(table_hbm.at[idx_v.at[b].at[j]],
                                 rows_v.at[b].at[pl.ds(j * G, G)],
                                 gsems[b])

        for b in range(NBUF):
            start_idx(b, b)

        def outer(i, carry):
            c0 = i * NBUF
            for b in range(NBUF):
                pltpu.make_async_copy(idx_hbm.at[pl.ds(0, K)],
                                      idx_v.at[b], isems[b]).wait()

                @pl.when(i > 0)
                def _():
                    pltpu.make_async_copy(
                        rows_v.at[b],
                        out_hbm.at[pl.ds(0, CH)], wsems[b]).wait()

                start_gathers(b)

            for b in range(NBUF):
                off = pl.multiple_of(base + (c0 + b) * CH, CH)
                pltpu.make_async_copy(
                    table_hbm.at[pl.ds(0, CH)], rows_v.at[b],
                    gsems[b]).wait()
                pltpu.async_copy(rows_v.at[b],
                                 out_hbm.at[pl.ds(off, CH)], wsems[b])

                @pl.when(i < n_outer - 1)
                def _():
                    start_idx(c0 + NBUF + b, b)

            return carry

        lax.fori_loop(0, n_outer, outer, 0)

        for b in range(NBUF):
            pltpu.make_async_copy(rows_v.at[b],
                                  out_hbm.at[pl.ds(0, CH)], wsems[b]).wait()

    return gather_kernel(table, idx2d)


def _tc_dist(xe, ye, seg):
    R = xe.shape[0]
    BLK = 1024
    dn = (((0,), (1,)), ((), ()))

    def body(xe_ref, ye_ref, m_ref, o_ref):
        ex = xe_ref[...]
        ey = ye_ref[...]
        m = m_ref[...]
        pr = jax.lax.Precision.DEFAULT
        f32 = jnp.float32
        sx = jax.lax.dot_general(m, ex * ex, dn, precision=pr,
                                 preferred_element_type=f32)
        sy = jax.lax.dot_general(m, ey * ey, dn, precision=pr,
                                 preferred_element_type=f32)
        pxy = jax.lax.dot_general(m, ex * ey, dn, precision=pr,
                                  preferred_element_type=f32)
        nx = jnp.sqrt(sx + 1e-24)
        ny = jnp.sqrt(sy + 1e-24)
        scx = jnp.where(nx > MAX_NORM, MAX_NORM / nx, 1.0)
        scy = jnp.where(ny > MAX_NORM, MAX_NORM / ny, 1.0)
        ax = sx * scx * scx
        ay = sy * scy * scy
        dd2 = jnp.clip(ax + ay - 2.0 * (scx * scy) * pxy, EPS, None)
        nx2 = jnp.clip(ax, EPS, None)
        ny2 = jnp.clip(ay, EPS, None)
        z = dd2 / ((1.0 - nx2) * (1.0 - ny2))
        t = 1.0 + 2.0 * z
        o_ref[...] = jnp.log(t + jnp.sqrt(t * t - 1.0))

    return pl.pallas_call(
        body,
        grid=(R // BLK,),
        in_specs=[
            pl.BlockSpec((BLK, 128), lambda i: (i, 0)),
            pl.BlockSpec((BLK, 128), lambda i: (i, 0)),
            pl.BlockSpec((128, 8), lambda i: (0, 0)),
        ],
        out_specs=pl.BlockSpec((8, BLK), lambda i: (0, i)),
        out_shape=jax.ShapeDtypeStruct((8, R), jnp.float32),
    )(xe, ye, seg)


def kernel(x, y, table):
    B, L = x.shape
    N = B * L
    idx = jnp.concatenate([x.reshape(-1), y.reshape(-1)]).astype(jnp.int32)
    idx2d = idx.reshape(-1, 128)
    g = _sc_gather(table, idx2d)
    xe = g[:N].reshape(N // 8, 128)
    ye = g[N:].reshape(N // 8, 128)
    seg = jnp.asarray(
        np.kron(np.eye(8, dtype=np.float32), np.ones((D, 1), np.float32)))
    dist8 = _tc_dist(xe, ye, seg)
    return dist8.T.reshape(B, L)


# speedup vs baseline: 2.9810x; 2.9810x over previous
import functools

import jax
import jax.numpy as jnp
import numpy as np
from jax import lax
from jax.experimental import pallas as pl
from jax.experimental.pallas import tpu as pltpu
from jax.experimental.pallas import tpu_sc as plsc

D = 16
EPS = 1e-05
MAX_NORM = 1.0 - EPS


def _sc_gather(table, idx2d):
    n_rows, G = idx2d.shape
    M = n_rows * G
    info = plsc.get_sparse_core_info()
    NC, NS = info.num_cores, info.num_subcores
    NW = NC * NS
    NBUF = 2
    K = 8
    CH = K * G
    per_w = M // NW
    n_chunks = per_w // CH
    n_outer = n_chunks // NBUF

    mesh = plsc.VectorSubcoreMesh(core_axis_name="c", subcore_axis_name="s")

    @functools.partial(
        pl.kernel,
        mesh=mesh,
        out_type=jax.ShapeDtypeStruct((M, D), jnp.float32),
        scratch_types=[
            pltpu.VMEM((NBUF, K, G), jnp.int32),
            pltpu.VMEM((NBUF, CH, D), jnp.float32),
            [pltpu.SemaphoreType.DMA] * NBUF,
            [pltpu.SemaphoreType.DMA] * NBUF,
            [pltpu.SemaphoreType.DMA] * NBUF,
        ],
        compiler_params=pltpu.CompilerParams(use_tc_tiling_on_sc=False),
    )
    def gather_kernel(table_hbm, idx_hbm, out_hbm, idx_v, rows_v,
                      isems, gsems, wsems):
        wid = lax.axis_index("s") * NC + lax.axis_index("c")
        base = wid * per_w

        def idx_row(c):
            return pl.multiple_of((base // G) + c * K, K)

        def start_idx(c, b):
            pltpu.async_copy(idx_hbm.at[pl.ds(idx_row(c), K)],
                             idx_v.at[b], isems[b])

        def start_gathers(b):
            for j in range(K):
                pltpu.async_copy(table_hbm.at[idx_v.at[b].at[j]],
                                 rows_v.at[b].at[pl.ds(j * G, G)],
                                 gsems[b])

        for b in range(NBUF):
            start_idx(b, b)

        def outer(i, carry):
            c0 = i * NBUF
            for b in range(NBUF):
                pltpu.make_async_copy(idx_hbm.at[pl.ds(0, K)],
                                      idx_v.at[b], isems[b]).wait()

                @pl.when(i > 0)
                def _():
                    pltpu.make_async_copy(
                        rows_v.at[b],
                        out_hbm.at[pl.ds(0, CH)], wsems[b]).wait()

                start_gathers(b)

            for b in range(NBUF):
                off = pl.multiple_of(base + (c0 + b) * CH, CH)
                pltpu.make_async_copy(
                    table_hbm.at[pl.ds(0, CH)], rows_v.at[b],
                    gsems[b]).wait()
                pltpu.async_copy(rows_v.at[b],
                                 out_hbm.at[pl.ds(off, CH)], wsems[b])

                @pl.when(i < n_outer - 1)
                def _():
                    start_idx(c0 + NBUF + b, b)

            return carry

        lax.fori_loop(0, n_outer, outer, 0)

        for b in range(NBUF):
            pltpu.make_async_copy(rows_v.at[b],
                                  out_hbm.at[pl.ds(0, CH)], wsems[b]).wait()

    return gather_kernel(table, idx2d)


def _tc_dist(g128, seg):
    R = g128.shape[0] // 2
    BLK = 1024
    half = R // BLK
    dn = (((0,), (1,)), ((), ()))

    def body(xe_ref, ye_ref, m_ref, o_ref):
        ex = xe_ref[...]
        ey = ye_ref[...]
        m = m_ref[...]
        pr = jax.lax.Precision.DEFAULT
        f32 = jnp.float32
        sx = jax.lax.dot_general(m, ex * ex, dn, precision=pr,
                                 preferred_element_type=f32)
        sy = jax.lax.dot_general(m, ey * ey, dn, precision=pr,
                                 preferred_element_type=f32)
        pxy = jax.lax.dot_general(m, ex * ey, dn, precision=pr,
                                  preferred_element_type=f32)
        nx = jnp.sqrt(sx + 1e-24)
        ny = jnp.sqrt(sy + 1e-24)
        scx = jnp.where(nx > MAX_NORM, MAX_NORM / nx, 1.0)
        scy = jnp.where(ny > MAX_NORM, MAX_NORM / ny, 1.0)
        ax = sx * scx * scx
        ay = sy * scy * scy
        dd2 = jnp.clip(ax + ay - 2.0 * (scx * scy) * pxy, EPS, None)
        nx2 = jnp.clip(ax, EPS, None)
        ny2 = jnp.clip(ay, EPS, None)
        z = dd2 / ((1.0 - nx2) * (1.0 - ny2))
        t = 1.0 + 2.0 * z
        o_ref[...] = jnp.log(t + jnp.sqrt(t * t - 1.0))

    return pl.pallas_call(
        body,
        grid=(R // BLK,),
        in_specs=[
            pl.BlockSpec((BLK, 128), lambda i: (i, 0)),
            pl.BlockSpec((BLK, 128), lambda i, h=half: (i + h, 0)),
            pl.BlockSpec((128, 8), lambda i: (0, 0)),
        ],
        out_specs=pl.BlockSpec((8, BLK), lambda i: (0, i)),
        out_shape=jax.ShapeDtypeStruct((8, R), jnp.float32),
    )(g128, g128, seg)


def _tc_reformat(tt):
    Dd, V = tt.shape
    BLK = 8192
    RO = BLK * Dd // 128

    def body(t_ref, o_ref):
        blk = t_ref[...]
        C = jax.lax.transpose(blk, (1, 0))
        cols = [jax.lax.slice(C, (a, 0), (BLK, Dd), (8, 1))
                for a in range(8)]
        o_ref[...] = jnp.concatenate(cols, axis=1)

    return pl.pallas_call(
        body,
        grid=((V + BLK - 1) // BLK,),
        in_specs=[pl.BlockSpec((Dd, BLK), lambda i: (0, i))],
        out_specs=pl.BlockSpec((RO, 128), lambda i: (i, 0)),
        out_shape=jax.ShapeDtypeStruct((V * Dd // 128, 128), jnp.float32),
    )(tt)


def kernel(x, y, table):
    B, L = x.shape
    N = B * L
    xt = x.T.reshape(-1)
    yt = y.T.reshape(-1)
    idx = jnp.concatenate([xt, yt]).astype(jnp.int32)
    idx2d = idx.reshape(-1, 128)
    g = _sc_gather(table, idx2d)
    g128 = g.reshape(2 * N // 8, 128)
    seg = jnp.asarray(
        np.kron(np.eye(8, dtype=np.float32), np.ones((D, 1), np.float32)))
    dist8 = _tc_dist(g128, seg)
    return dist8.T.reshape(L, B).T


# speedup vs baseline: 4.3470x; 1.4582x over previous
import functools

import jax
import jax.numpy as jnp
import numpy as np
from jax import lax
from jax.experimental import pallas as pl
from jax.experimental.pallas import tpu as pltpu
from jax.experimental.pallas import tpu_sc as plsc

D = 16
EPS = 1e-05
MAX_NORM = 1.0 - EPS


def _sc_gather(table, idx2d):
    n_rows, G = idx2d.shape
    M = n_rows * G
    info = plsc.get_sparse_core_info()
    NC, NS = info.num_cores, info.num_subcores
    NW = NC * NS
    NBUF = 2
    K = 8
    CH = K * G
    per_w = M // NW
    n_chunks = per_w // CH
    n_outer = n_chunks // NBUF

    mesh = plsc.VectorSubcoreMesh(core_axis_name="c", subcore_axis_name="s")

    @functools.partial(
        pl.kernel,
        mesh=mesh,
        out_type=jax.ShapeDtypeStruct((M, D), jnp.float32),
        scratch_types=[
            pltpu.VMEM((NBUF, K, G), jnp.int32),
            pltpu.VMEM((NBUF, CH, D), jnp.float32),
            [pltpu.SemaphoreType.DMA] * NBUF,
            [pltpu.SemaphoreType.DMA] * NBUF,
            [pltpu.SemaphoreType.DMA] * NBUF,
        ],
        compiler_params=pltpu.CompilerParams(use_tc_tiling_on_sc=False),
    )
    def gather_kernel(table_hbm, idx_hbm, out_hbm, idx_v, rows_v,
                      isems, gsems, wsems):
        wid = lax.axis_index("s") * NC + lax.axis_index("c")
        base = wid * per_w

        def idx_row(c):
            return pl.multiple_of((base // G) + c * K, K)

        def start_idx(c, b):
            pltpu.async_copy(idx_hbm.at[pl.ds(idx_row(c), K)],
                             idx_v.at[b], isems[b])

        def start_gathers(b):
            for j in range(K):
                pltpu.async_copy(table_hbm.at[idx_v.at[b].at[j]],
                                 rows_v.at[b].at[pl.ds(j * G, G)],
                                 gsems[b])

        for b in range(NBUF):
            start_idx(b, b)

        def outer(i, carry):
            c0 = i * NBUF
            for b in range(NBUF):
                pltpu.make_async_copy(idx_hbm.at[pl.ds(0, K)],
                                      idx_v.at[b], isems[b]).wait()

                @pl.when(i > 0)
                def _():
                    pltpu.make_async_copy(
                        rows_v.at[b],
                        out_hbm.at[pl.ds(0, CH)], wsems[b]).wait()

                start_gathers(b)

            for b in range(NBUF):
                off = pl.multiple_of(base + (c0 + b) * CH, CH)
                pltpu.make_async_copy(
                    table_hbm.at[pl.ds(0, CH)], rows_v.at[b],
                    gsems[b]).wait()
                pltpu.async_copy(rows_v.at[b],
                                 out_hbm.at[pl.ds(off, CH)], wsems[b])

                @pl.when(i < n_outer - 1)
                def _():
                    start_idx(c0 + NBUF + b, b)

            return carry

        lax.fori_loop(0, n_outer, outer, 0)

        for b in range(NBUF):
            pltpu.make_async_copy(rows_v.at[b],
                                  out_hbm.at[pl.ds(0, CH)], wsems[b]).wait()

    return gather_kernel(table, idx2d)


def _tc_dist(g128, seg):
    R = g128.shape[0] // 2
    BLK = 1024
    half = R // BLK
    dn = (((0,), (1,)), ((), ()))

    def body(xe_ref, ye_ref, m_ref, o_ref):
        ex = xe_ref[...]
        ey = ye_ref[...]
        m = m_ref[...]
        pr = jax.lax.Precision.DEFAULT
        f32 = jnp.float32
        sx = jax.lax.dot_general(m, ex * ex, dn, precision=pr,
                                 preferred_element_type=f32)
        sy = jax.lax.dot_general(m, ey * ey, dn, precision=pr,
                                 preferred_element_type=f32)
        pxy = jax.lax.dot_general(m, ex * ey, dn, precision=pr,
                                  preferred_element_type=f32)
        nx = jnp.sqrt(sx + 1e-24)
        ny = jnp.sqrt(sy + 1e-24)
        scx = jnp.where(nx > MAX_NORM, MAX_NORM / nx, 1.0)
        scy = jnp.where(ny > MAX_NORM, MAX_NORM / ny, 1.0)
        ax = sx * scx * scx
        ay = sy * scy * scy
        dd2 = jnp.clip(ax + ay - 2.0 * (scx * scy) * pxy, EPS, None)
        nx2 = jnp.clip(ax, EPS, None)
        ny2 = jnp.clip(ay, EPS, None)
        z = dd2 / ((1.0 - nx2) * (1.0 - ny2))
        t = 1.0 + 2.0 * z
        o_ref[...] = jnp.log(t + jnp.sqrt(t * t - 1.0))

    return pl.pallas_call(
        body,
        grid=(R // BLK,),
        in_specs=[
            pl.BlockSpec((BLK, 128), lambda i: (i, 0)),
            pl.BlockSpec((BLK, 128), lambda i, h=half: (i + h, 0)),
            pl.BlockSpec((128, 8), lambda i: (0, 0)),
        ],
        out_specs=pl.BlockSpec((8, BLK), lambda i: (0, i)),
        out_shape=jax.ShapeDtypeStruct((8, R), jnp.float32),
    )(g128, g128, seg)


def _sc_transpose(tt, tail):
    Dd, V = tt.shape
    info = plsc.get_sparse_core_info()
    NC, NS = info.num_cores, info.num_subcores
    NW = NC * NS
    C = 512
    TPW = 244
    CPW = TPW * 128 // C
    GR = C // 16
    extra_c0 = NW * TPW * 128
    tail_row0 = (V // 8) - 8

    mesh = plsc.VectorSubcoreMesh(core_axis_name="c", subcore_axis_name="s")

    @functools.partial(
        pl.kernel,
        mesh=mesh,
        out_type=jax.ShapeDtypeStruct((V * Dd // 128, 128), jnp.float32),
        scratch_types=[
            pltpu.VMEM((Dd, C), jnp.float32),
            pltpu.VMEM((C * Dd // 128, 128), jnp.float32),
            pltpu.VMEM((8, 128), jnp.float32),
        ],
        compiler_params=pltpu.CompilerParams(use_tc_tiling_on_sc=True, needs_layout_passes=False),
    )
    def transpose_kernel(tt_hbm, tail_hbm, out_hbm, pin, rowbuf, tailbuf):
        wid = lax.axis_index("s") * NC + lax.axis_index("c")
        iota = jax.lax.iota(jnp.int32, 16)
        sub = iota >> 3
        lane0 = (iota & 7) << 4

        def do_chunk(c0):
            pltpu.sync_copy(tt_hbm.at[:, pl.ds(c0, C)], pin)

            def grp(g, carry):
                rows = 2 * g + sub
                for d in range(Dd):
                    v = pin[d, pl.ds(g * 16, 16)]
                    plsc.store_scatter(rowbuf, [rows, lane0 + d], v)
                return carry

            lax.fori_loop(0, GR, grp, 0)
            pltpu.sync_copy(
                rowbuf, out_hbm.at[pl.ds(pl.multiple_of(c0 // 8, C // 8),
                                         C * Dd // 128)])

        def chunk_loop(i, carry):
            do_chunk(pl.multiple_of(wid * TPW * 128 + i * C, C))
            return carry

        lax.fori_loop(0, CPW, chunk_loop, 0)

        @pl.when(wid == 0)
        def _():
            do_chunk(extra_c0)

        @pl.when(wid == 1)
        def _():
            pltpu.sync_copy(tail_hbm, tailbuf)
            pltpu.sync_copy(tailbuf, out_hbm.at[pl.ds(tail_row0, 8)])

    return transpose_kernel(tt, tail)


def kernel(x, y, table):
    B, L = x.shape
    N = B * L
    xt = x.T.reshape(-1)
    yt = y.T.reshape(-1)
    idx = jnp.concatenate([xt, yt]).astype(jnp.int32)
    idx2d = idx.reshape(-1, 128)
    V = table.shape[0]
    tail = jax.lax.slice(table, (V - 64, 0), (V, D)).reshape(8, 128)
    tlin = _sc_transpose(table.T, tail).reshape(V, D)
    g = _sc_gather(tlin, idx2d)
    g128 = g.reshape(2 * N // 8, 128)
    seg = jnp.asarray(
        np.kron(np.eye(8, dtype=np.float32), np.ones((D, 1), np.float32)))
    dist8 = _tc_dist(g128, seg)
    return dist8.T.reshape(L, B).T


# speedup vs baseline: 4.8426x; 1.1140x over previous
import functools

import jax
import jax.numpy as jnp
import numpy as np
from jax import lax
from jax.experimental import pallas as pl
from jax.experimental.pallas import tpu as pltpu
from jax.experimental.pallas import tpu_sc as plsc

D = 16
EPS = 1e-05
MAX_NORM = 1.0 - EPS


def _sc_gather(table, idx8):
    J, Q = idx8.shape
    G = 128
    info = plsc.get_sparse_core_info()
    NC, NS = info.num_cores, info.num_subcores
    NW = NC * NS
    NBUF = 2
    CH = J * G
    q_per_w = Q // NW
    n_chunks = q_per_w // G
    n_outer = n_chunks // NBUF

    mesh = plsc.VectorSubcoreMesh(core_axis_name="c", subcore_axis_name="s")

    @functools.partial(
        pl.kernel,
        mesh=mesh,
        out_type=jax.ShapeDtypeStruct((Q, J * D), jnp.float32),
        scratch_types=[
            pltpu.VMEM((NBUF, J, G), jnp.int32),
            pltpu.VMEM((NBUF, CH, D), jnp.float32),
            [pltpu.SemaphoreType.DMA] * NBUF,
            [pltpu.SemaphoreType.DMA] * NBUF,
            [pltpu.SemaphoreType.DMA] * NBUF,
        ],
        compiler_params=pltpu.CompilerParams(use_tc_tiling_on_sc=False),
    )
    def gather_kernel(table_hbm, idx_hbm, out_hbm, idx_v, rows_v,
                      isems, gsems, wsems):
        wid = lax.axis_index("s") * NC + lax.axis_index("c")
        qbase = wid * q_per_w

        def start_idx(c, b):
            col = pl.multiple_of(qbase + c * G, G)
            pltpu.async_copy(idx_hbm.at[:, pl.ds(col, G)],
                             idx_v.at[b], isems[b])

        def start_gathers(b):
            for j in range(J):
                pltpu.async_copy(table_hbm.at[idx_v.at[b].at[j]],
                                 rows_v.at[b].at[pl.ds(j * G, G)],
                                 gsems[b])

        def start_writeback(b, q0):
            for j in range(J):
                pltpu.async_copy(rows_v.at[b].at[pl.ds(j * G, G)],
                                 out_hbm.at[pl.ds(q0, G), pl.ds(j * D, D)],
                                 wsems[b])

        def drain_writeback(b):
            for j in range(J):
                pltpu.make_async_copy(rows_v.at[b].at[pl.ds(j * G, G)],
                                      out_hbm.at[pl.ds(0, G), pl.ds(j * D, D)],
                                      wsems[b]).wait()

        for b in range(NBUF):
            start_idx(b, b)

        def outer(i, carry):
            c0 = i * NBUF
            for b in range(NBUF):
                pltpu.make_async_copy(idx_hbm.at[:, pl.ds(0, G)],
                                      idx_v.at[b], isems[b]).wait()

                @pl.when(i > 0)
                def _():
                    drain_writeback(b)

                start_gathers(b)

            for b in range(NBUF):
                q0 = pl.multiple_of(qbase + (c0 + b) * G, G)
                pltpu.make_async_copy(
                    table_hbm.at[pl.ds(0, CH)], rows_v.at[b],
                    gsems[b]).wait()
                start_writeback(b, q0)

                @pl.when(i < n_outer - 1)
                def _():
                    start_idx(c0 + NBUF + b, b)

            return carry

        lax.fori_loop(0, n_outer, outer, 0)

        for b in range(NBUF):
            drain_writeback(b)

    return gather_kernel(table, idx8)


def _tc_dist(g128, seg):
    R = g128.shape[0] // 2
    BLK = 1024
    half = R // BLK
    dn = (((0,), (1,)), ((), ()))

    def body(xe_ref, ye_ref, m_ref, o_ref):
        ex = xe_ref[...]
        ey = ye_ref[...]
        m = m_ref[...]
        pr = jax.lax.Precision.DEFAULT
        f32 = jnp.float32
        sx = jax.lax.dot_general(m, ex * ex, dn, precision=pr,
                                 preferred_element_type=f32)
        sy = jax.lax.dot_general(m, ey * ey, dn, precision=pr,
                                 preferred_element_type=f32)
        pxy = jax.lax.dot_general(m, ex * ey, dn, precision=pr,
                                  preferred_element_type=f32)
        nx = jnp.sqrt(sx + 1e-24)
        ny = jnp.sqrt(sy + 1e-24)
        scx = jnp.where(nx > MAX_NORM, MAX_NORM / nx, 1.0)
        scy = jnp.where(ny > MAX_NORM, MAX_NORM / ny, 1.0)
        ax = sx * scx * scx
        ay = sy * scy * scy
        dd2 = jnp.clip(ax + ay - 2.0 * (scx * scy) * pxy, EPS, None)
        nx2 = jnp.clip(ax, EPS, None)
        ny2 = jnp.clip(ay, EPS, None)
        z = dd2 / ((1.0 - nx2) * (1.0 - ny2))
        t = 1.0 + 2.0 * z
        o_ref[...] = jnp.log(t + jnp.sqrt(t * t - 1.0))

    return pl.pallas_call(
        body,
        grid=(R // BLK,),
        in_specs=[
            pl.BlockSpec((BLK, 128), lambda i: (i, 0)),
            pl.BlockSpec((BLK, 128), lambda i, h=half: (i + h, 0)),
            pl.BlockSpec((128, 8), lambda i: (0, 0)),
        ],
        out_specs=pl.BlockSpec((8, BLK), lambda i: (0, i)),
        out_shape=jax.ShapeDtypeStruct((8, R), jnp.float32),
    )(g128, g128, seg)


def _sc_transpose(tt, tail):
    Dd, V = tt.shape
    info = plsc.get_sparse_core_info()
    NC, NS = info.num_cores, info.num_subcores
    NW = NC * NS
    C = 512
    TPW = 244
    CPW = TPW * 128 // C
    GR = C // 16
    extra_c0 = NW * TPW * 128
    tail_row0 = (V // 8) - 8

    mesh = plsc.VectorSubcoreMesh(core_axis_name="c", subcore_axis_name="s")

    @functools.partial(
        pl.kernel,
        mesh=mesh,
        out_type=jax.ShapeDtypeStruct((V * Dd // 128, 128), jnp.float32),
        scratch_types=[
            pltpu.VMEM((Dd, C), jnp.float32),
            pltpu.VMEM((C * Dd // 128, 128), jnp.float32),
            pltpu.VMEM((8, 128), jnp.float32),
        ],
        compiler_params=pltpu.CompilerParams(use_tc_tiling_on_sc=True, needs_layout_passes=False),
    )
    def transpose_kernel(tt_hbm, tail_hbm, out_hbm, pin, rowbuf, tailbuf):
        wid = lax.axis_index("s") * NC + lax.axis_index("c")
        iota = jax.lax.iota(jnp.int32, 16)

        def do_chunk(c0):
            pltpu.sync_copy(tt_hbm.at[:, pl.ds(c0, C)], pin)

            def grp(g, carry):
                col = g * 16 + iota
                for d0 in range(Dd):
                    dd = (d0 + iota) & 15
                    v = plsc.load_gather(pin, [dd, col])
                    flat = (col << 4) + dd
                    plsc.store_scatter(rowbuf, [flat >> 7, flat & 127], v)
                return carry

            lax.fori_loop(0, GR, grp, 0)
            pltpu.sync_copy(
                rowbuf, out_hbm.at[pl.ds(pl.multiple_of(c0 // 8, C // 8),
                                         C * Dd // 128)])

        def chunk_loop(i, carry):
            do_chunk(pl.multiple_of(wid * TPW * 128 + i * C, C))
            return carry

        lax.fori_loop(0, CPW, chunk_loop, 0)

        @pl.when(wid == 0)
        def _():
            do_chunk(extra_c0)

        @pl.when(wid == 1)
        def _():
            pltpu.sync_copy(tail_hbm, tailbuf)
            pltpu.sync_copy(tailbuf, out_hbm.at[pl.ds(tail_row0, 8)])

    return transpose_kernel(tt, tail)


def kernel(x, y, table):
    B, L = x.shape
    N = B * L
    xt8 = x.T.reshape(8, N // 8)
    yt8 = y.T.reshape(8, N // 8)
    idx8 = jnp.concatenate([xt8, yt8], axis=1).astype(jnp.int32)
    V = table.shape[0]
    tail = jax.lax.slice(table, (V - 64, 0), (V, D)).reshape(8, 128)
    tlin = _sc_transpose(table.T, tail).reshape(V, D)
    g128 = _sc_gather(tlin, idx8)
    seg = jnp.asarray(
        np.kron(np.eye(8, dtype=np.float32), np.ones((D, 1), np.float32)))
    dist8 = _tc_dist(g128, seg)
    return dist8.reshape(L, B).T


# speedup vs baseline: 5.9520x; 1.2291x over previous
import functools

import jax
import jax.numpy as jnp
import numpy as np
from jax import lax
from jax.experimental import pallas as pl
from jax.experimental.pallas import tpu as pltpu
from jax.experimental.pallas import tpu_sc as plsc

D = 16
EPS = 1e-05
MAX_NORM = 1.0 - EPS


def _sc_gather(table, idx8):
    J, Q = idx8.shape
    G = 128
    info = plsc.get_sparse_core_info()
    NC, NS = info.num_cores, info.num_subcores
    NW = NC * NS
    NBUF = 2
    CH = J * G
    q_per_w = Q // NW
    n_chunks = q_per_w // G
    n_outer = n_chunks // NBUF

    mesh = plsc.VectorSubcoreMesh(core_axis_name="c", subcore_axis_name="s")

    @functools.partial(
        pl.kernel,
        mesh=mesh,
        out_type=jax.ShapeDtypeStruct((Q, J * D), jnp.float32),
        scratch_types=[
            pltpu.VMEM((NBUF, J, G), jnp.int32),
            pltpu.VMEM((NBUF, CH, D), jnp.float32),
            [pltpu.SemaphoreType.DMA] * NBUF,
            [pltpu.SemaphoreType.DMA] * NBUF,
            [pltpu.SemaphoreType.DMA] * NBUF,
        ],
        compiler_params=pltpu.CompilerParams(use_tc_tiling_on_sc=False),
    )
    def gather_kernel(table_hbm, idx_hbm, out_hbm, idx_v, rows_v,
                      isems, gsems, wsems):
        wid = lax.axis_index("s") * NC + lax.axis_index("c")
        qbase = wid * q_per_w

        def start_idx(c, b):
            col = pl.multiple_of(qbase + c * G, G)
            pltpu.async_copy(idx_hbm.at[:, pl.ds(col, G)],
                             idx_v.at[b], isems[b])

        def start_gathers(b):
            for j in range(J):
                pltpu.async_copy(table_hbm.at[idx_v.at[b].at[j]],
                                 rows_v.at[b].at[pl.ds(j * G, G)],
                                 gsems[b])

        def start_writeback(b, q0):
            for j in range(J):
                pltpu.async_copy(rows_v.at[b].at[pl.ds(j * G, G)],
                                 out_hbm.at[pl.ds(q0, G), pl.ds(j * D, D)],
                                 wsems[b])

        def drain_writeback(b):
            for j in range(J):
                pltpu.make_async_copy(rows_v.at[b].at[pl.ds(j * G, G)],
                                      out_hbm.at[pl.ds(0, G), pl.ds(j * D, D)],
                                      wsems[b]).wait()

        for b in range(NBUF):
            start_idx(b, b)

        def outer(i, carry):
            c0 = i * NBUF
            for b in range(NBUF):
                pltpu.make_async_copy(idx_hbm.at[:, pl.ds(0, G)],
                                      idx_v.at[b], isems[b]).wait()

                @pl.when(i > 0)
                def _():
                    drain_writeback(b)

                start_gathers(b)

            for b in range(NBUF):
                q0 = pl.multiple_of(qbase + (c0 + b) * G, G)
                pltpu.make_async_copy(
                    table_hbm.at[pl.ds(0, CH)], rows_v.at[b],
                    gsems[b]).wait()
                start_writeback(b, q0)

                @pl.when(i < n_outer - 1)
                def _():
                    start_idx(c0 + NBUF + b, b)

            return carry

        lax.fori_loop(0, n_outer, outer, 0)

        for b in range(NBUF):
            drain_writeback(b)

    return gather_kernel(table, idx8)


def _tc_dist(g128, seg):
    R = g128.shape[0] // 2
    BLK = 1024
    half = R // BLK
    dn = (((0,), (1,)), ((), ()))

    def body(xe_ref, ye_ref, m_ref, o_ref):
        ex = xe_ref[...]
        ey = ye_ref[...]
        m = m_ref[...]
        pr = jax.lax.Precision.DEFAULT
        f32 = jnp.float32
        sx = jax.lax.dot_general(m, ex * ex, dn, precision=pr,
                                 preferred_element_type=f32)
        sy = jax.lax.dot_general(m, ey * ey, dn, precision=pr,
                                 preferred_element_type=f32)
        pxy = jax.lax.dot_general(m, ex * ey, dn, precision=pr,
                                  preferred_element_type=f32)
        nx = jnp.sqrt(sx + 1e-24)
        ny = jnp.sqrt(sy + 1e-24)
        scx = jnp.where(nx > MAX_NORM, MAX_NORM / nx, 1.0)
        scy = jnp.where(ny > MAX_NORM, MAX_NORM / ny, 1.0)
        ax = sx * scx * scx
        ay = sy * scy * scy
        dd2 = jnp.clip(ax + ay - 2.0 * (scx * scy) * pxy, EPS, None)
        nx2 = jnp.clip(ax, EPS, None)
        ny2 = jnp.clip(ay, EPS, None)
        z = dd2 / ((1.0 - nx2) * (1.0 - ny2))
        t = 1.0 + 2.0 * z
        o_ref[...] = jnp.log(t + jnp.sqrt(t * t - 1.0))

    return pl.pallas_call(
        body,
        grid=(R // BLK,),
        in_specs=[
            pl.BlockSpec((BLK, 128), lambda i: (i, 0)),
            pl.BlockSpec((BLK, 128), lambda i, h=half: (i + h, 0)),
            pl.BlockSpec((128, 8), lambda i: (0, 0)),
        ],
        out_specs=pl.BlockSpec((8, BLK), lambda i: (0, i)),
        out_shape=jax.ShapeDtypeStruct((8, R), jnp.float32),
    )(g128, g128, seg)


def _sc_transpose(tt, tail):
    Dd, V = tt.shape
    info = plsc.get_sparse_core_info()
    NC, NS = info.num_cores, info.num_subcores
    NW = NC * NS
    C = 256
    RO = C * Dd // 128
    TPW = 244
    CPW = TPW * 128 // C
    NBUF = 2
    n_outer = CPW // NBUF
    GR = C // 16
    extra_c0 = NW * TPW * 128
    tail_row0 = (V // 8) - 8

    mesh = plsc.VectorSubcoreMesh(core_axis_name="c", subcore_axis_name="s")

    @functools.partial(
        pl.kernel,
        mesh=mesh,
        out_type=jax.ShapeDtypeStruct((V * Dd // 128, 128), jnp.float32),
        scratch_types=[
            pltpu.VMEM((NBUF, Dd, C), jnp.float32),
            pltpu.VMEM((NBUF, RO, 128), jnp.float32),
            pltpu.VMEM((8, 128), jnp.float32),
            [pltpu.SemaphoreType.DMA] * NBUF,
            [pltpu.SemaphoreType.DMA] * NBUF,
        ],
        compiler_params=pltpu.CompilerParams(use_tc_tiling_on_sc=True, needs_layout_passes=False),
    )
    def transpose_kernel(tt_hbm, tail_hbm, out_hbm, pin, rowbuf, tailbuf,
                         isems, osems):
        wid = lax.axis_index("s") * NC + lax.axis_index("c")
        iota = jax.lax.iota(jnp.int32, 16)
        base = wid * TPW * 128

        def start_in(c0, b):
            pltpu.async_copy(tt_hbm.at[:, pl.ds(c0, C)], pin.at[b],
                             isems[b])

        def do_compute(b):
            def grp(g, carry):
                col = g * 16 + iota
                for d0 in range(Dd):
                    dd = (d0 + iota) & 15
                    v = plsc.load_gather(pin.at[b], [dd, col])
                    flat = (col << 4) + dd
                    plsc.store_scatter(rowbuf.at[b],
                                       [flat >> 7, flat & 127], v)
                return carry

            lax.fori_loop(0, GR, grp, 0)

        def start_out(c0, b):
            pltpu.async_copy(
                rowbuf.at[b],
                out_hbm.at[pl.ds(pl.multiple_of(c0 // 8, RO), RO)],
                osems[b])

        def do_chunk_sync(c0):
            pltpu.sync_copy(tt_hbm.at[:, pl.ds(c0, C)], pin.at[0])
            do_compute(0)
            pltpu.sync_copy(
                rowbuf.at[0],
                out_hbm.at[pl.ds(pl.multiple_of(c0 // 8, RO), RO)])

        for b in range(NBUF):
            start_in(base + b * C, b)

        def outer(i, carry):
            c0 = i * NBUF
            for b in range(NBUF):
                pltpu.make_async_copy(tt_hbm.at[:, pl.ds(0, C)],
                                      pin.at[b], isems[b]).wait()

                @pl.when(i > 0)
                def _():
                    pltpu.make_async_copy(
                        rowbuf.at[b], out_hbm.at[pl.ds(0, RO)],
                        osems[b]).wait()

                do_compute(b)
                start_out(pl.multiple_of(base + (c0 + b) * C, C), b)

                @pl.when(i < n_outer - 1)
                def _():
                    start_in(pl.multiple_of(base + (c0 + NBUF + b) * C, C),
                             b)

            return carry

        lax.fori_loop(0, n_outer, outer, 0)

        for b in range(NBUF):
            pltpu.make_async_copy(rowbuf.at[b], out_hbm.at[pl.ds(0, RO)],
                                  osems[b]).wait()

        @pl.when(wid == 0)
        def _():
            do_chunk_sync(extra_c0)
            do_chunk_sync(extra_c0 + C)

        @pl.when(wid == 1)
        def _():
            pltpu.sync_copy(tail_hbm, tailbuf)
            pltpu.sync_copy(tailbuf, out_hbm.at[pl.ds(tail_row0, 8)])

    return transpose_kernel(tt, tail)


def kernel(x, y, table):
    B, L = x.shape
    N = B * L
    xt8 = x.T.reshape(8, N // 8)
    yt8 = y.T.reshape(8, N // 8)
    idx8 = jnp.concatenate([xt8, yt8], axis=1).astype(jnp.int32)
    V = table.shape[0]
    tail = jax.lax.slice(table, (V - 64, 0), (V, D)).reshape(8, 128)
    tlin = _sc_transpose(table.T, tail).reshape(V, D)
    g128 = _sc_gather(tlin, idx8)
    seg = jnp.asarray(
        np.kron(np.eye(8, dtype=np.float32), np.ones((D, 1), np.float32)))
    dist8 = _tc_dist(g128, seg)
    return dist8.reshape(L, B).T


# speedup vs baseline: 6.1109x; 1.0267x over previous
import functools

import jax
import jax.numpy as jnp
import numpy as np
from jax import lax
from jax.experimental import pallas as pl
from jax.experimental.pallas import tpu as pltpu
from jax.experimental.pallas import tpu_sc as plsc

D = 16
EPS = 1e-05
MAX_NORM = 1.0 - EPS


def _sc_gather(table, idx8):
    J, Q = idx8.shape
    G = 128
    info = plsc.get_sparse_core_info()
    NC, NS = info.num_cores, info.num_subcores
    NW = NC * NS
    NBUF = 5
    CH = J * G
    q_per_w = Q // NW
    n_chunks = q_per_w // G
    n_outer = n_chunks // NBUF

    mesh = plsc.VectorSubcoreMesh(core_axis_name="c", subcore_axis_name="s")

    @functools.partial(
        pl.kernel,
        mesh=mesh,
        out_type=jax.ShapeDtypeStruct((Q, J * D), jnp.float32),
        scratch_types=[
            pltpu.VMEM((NBUF, J, G), jnp.int32),
            pltpu.VMEM((NBUF, CH, D), jnp.float32),
            [pltpu.SemaphoreType.DMA] * NBUF,
            [pltpu.SemaphoreType.DMA] * NBUF,
            [pltpu.SemaphoreType.DMA] * NBUF,
        ],
        compiler_params=pltpu.CompilerParams(use_tc_tiling_on_sc=False),
    )
    def gather_kernel(table_hbm, idx_hbm, out_hbm, idx_v, rows_v,
                      isems, gsems, wsems):
        wid = lax.axis_index("s") * NC + lax.axis_index("c")
        qbase = wid * q_per_w

        def start_idx(c, b):
            col = pl.multiple_of(qbase + c * G, G)
            pltpu.async_copy(idx_hbm.at[:, pl.ds(col, G)],
                             idx_v.at[b], isems[b])

        def start_gathers(b):
            for j in range(J):
                pltpu.async_copy(table_hbm.at[idx_v.at[b].at[j]],
                                 rows_v.at[b].at[pl.ds(j * G, G)],
                                 gsems[b])

        def start_writeback(b, q0):
            for j in range(J):
                pltpu.async_copy(rows_v.at[b].at[pl.ds(j * G, G)],
                                 out_hbm.at[pl.ds(q0, G), pl.ds(j * D, D)],
                                 wsems[b])

        def drain_writeback(b):
            for j in range(J):
                pltpu.make_async_copy(rows_v.at[b].at[pl.ds(j * G, G)],
                                      out_hbm.at[pl.ds(0, G), pl.ds(j * D, D)],
                                      wsems[b]).wait()

        for b in range(NBUF):
            start_idx(b, b)

        def outer(i, carry):
            c0 = i * NBUF
            for b in range(NBUF):
                pltpu.make_async_copy(idx_hbm.at[:, pl.ds(0, G)],
                                      idx_v.at[b], isems[b]).wait()

                @pl.when(i > 0)
                def _():
                    drain_writeback(b)

                start_gathers(b)

            for b in range(NBUF):
                q0 = pl.multiple_of(qbase + (c0 + b) * G, G)
                pltpu.make_async_copy(
                    table_hbm.at[pl.ds(0, CH)], rows_v.at[b],
                    gsems[b]).wait()
                start_writeback(b, q0)

                @pl.when(i < n_outer - 1)
                def _():
                    start_idx(c0 + NBUF + b, b)

            return carry

        lax.fori_loop(0, n_outer, outer, 0)

        for b in range(NBUF):
            drain_writeback(b)

    return gather_kernel(table, idx8)


def _tc_dist(g128, seg):
    R = g128.shape[0] // 2
    BLK = 1024
    half = R // BLK
    dn = (((0,), (1,)), ((), ()))

    def body(xe_ref, ye_ref, m_ref, o_ref):
        ex = xe_ref[...]
        ey = ye_ref[...]
        m = m_ref[...]
        pr = jax.lax.Precision.DEFAULT
        f32 = jnp.float32
        sx = jax.lax.dot_general(m, ex * ex, dn, precision=pr,
                                 preferred_element_type=f32)
        sy = jax.lax.dot_general(m, ey * ey, dn, precision=pr,
                                 preferred_element_type=f32)
        pxy = jax.lax.dot_general(m, ex * ey, dn, precision=pr,
                                  preferred_element_type=f32)
        nx = jnp.sqrt(sx + 1e-24)
        ny = jnp.sqrt(sy + 1e-24)
        scx = jnp.where(nx > MAX_NORM, MAX_NORM / nx, 1.0)
        scy = jnp.where(ny > MAX_NORM, MAX_NORM / ny, 1.0)
        ax = sx * scx * scx
        ay = sy * scy * scy
        dd2 = jnp.clip(ax + ay - 2.0 * (scx * scy) * pxy, EPS, None)
        nx2 = jnp.clip(ax, EPS, None)
        ny2 = jnp.clip(ay, EPS, None)
        z = dd2 / ((1.0 - nx2) * (1.0 - ny2))
        t = 1.0 + 2.0 * z
        o_ref[...] = jnp.log(t + jnp.sqrt(t * t - 1.0))

    return pl.pallas_call(
        body,
        grid=(R // BLK,),
        in_specs=[
            pl.BlockSpec((BLK, 128), lambda i: (i, 0)),
            pl.BlockSpec((BLK, 128), lambda i, h=half: (i + h, 0)),
            pl.BlockSpec((128, 8), lambda i: (0, 0)),
        ],
        out_specs=pl.BlockSpec((8, BLK), lambda i: (0, i)),
        out_shape=jax.ShapeDtypeStruct((8, R), jnp.float32),
    )(g128, g128, seg)


def _sc_transpose(tt, tail):
    Dd, V = tt.shape
    info = plsc.get_sparse_core_info()
    NC, NS = info.num_cores, info.num_subcores
    NW = NC * NS
    C = 256
    RO = C * Dd // 128
    TPW = 244
    CPW = TPW * 128 // C
    NBUF = 2
    n_outer = CPW // NBUF
    GR = C // 16
    extra_c0 = NW * TPW * 128
    tail_row0 = (V // 8) - 8

    mesh = plsc.VectorSubcoreMesh(core_axis_name="c", subcore_axis_name="s")

    @functools.partial(
        pl.kernel,
        mesh=mesh,
        out_type=jax.ShapeDtypeStruct((V * Dd // 128, 128), jnp.float32),
        scratch_types=[
            pltpu.VMEM((NBUF, Dd, C), jnp.float32),
            pltpu.VMEM((NBUF, RO, 128), jnp.float32),
            pltpu.VMEM((8, 128), jnp.float32),
            [pltpu.SemaphoreType.DMA] * NBUF,
            [pltpu.SemaphoreType.DMA] * NBUF,
        ],
        compiler_params=pltpu.CompilerParams(use_tc_tiling_on_sc=True, needs_layout_passes=False),
    )
    def transpose_kernel(tt_hbm, tail_hbm, out_hbm, pin, rowbuf, tailbuf,
                         isems, osems):
        wid = lax.axis_index("s") * NC + lax.axis_index("c")
        iota = jax.lax.iota(jnp.int32, 16)
        base = wid * TPW * 128

        def start_in(c0, b):
            pltpu.async_copy(tt_hbm.at[:, pl.ds(c0, C)], pin.at[b],
                             isems[b])

        def do_compute(b):
            def grp(g, carry):
                col = g * 16 + iota
                flat0 = col << 4
                for d in range(Dd):
                    v = pin.at[b][d, pl.ds(g * 16, 16)]
                    flat = flat0 + d
                    plsc.store_scatter(rowbuf.at[b],
                                       [flat >> 7, flat & 127], v)
                return carry

            lax.fori_loop(0, GR, grp, 0)

        def start_out(c0, b):
            pltpu.async_copy(
                rowbuf.at[b],
                out_hbm.at[pl.ds(pl.multiple_of(c0 // 8, RO), RO)],
                osems[b])

        def do_chunk_sync(c0):
            pltpu.sync_copy(tt_hbm.at[:, pl.ds(c0, C)], pin.at[0])
            do_compute(0)
            pltpu.sync_copy(
                rowbuf.at[0],
                out_hbm.at[pl.ds(pl.multiple_of(c0 // 8, RO), RO)])

        for b in range(NBUF):
            start_in(base + b * C, b)

        def outer(i, carry):
            c0 = i * NBUF
            for b in range(NBUF):
                pltpu.make_async_copy(tt_hbm.at[:, pl.ds(0, C)],
                                      pin.at[b], isems[b]).wait()

                @pl.when(i > 0)
                def _():
                    pltpu.make_async_copy(
                        rowbuf.at[b], out_hbm.at[pl.ds(0, RO)],
                        osems[b]).wait()

                do_compute(b)
                start_out(pl.multiple_of(base + (c0 + b) * C, C), b)

                @pl.when(i < n_outer - 1)
                def _():
                    start_in(pl.multiple_of(base + (c0 + NBUF + b) * C, C),
                             b)

            return carry

        lax.fori_loop(0, n_outer, outer, 0)

        for b in range(NBUF):
            pltpu.make_async_copy(rowbuf.at[b], out_hbm.at[pl.ds(0, RO)],
                                  osems[b]).wait()

        @pl.when(wid == 0)
        def _():
            do_chunk_sync(extra_c0)
            do_chunk_sync(extra_c0 + C)

        @pl.when(wid == 1)
        def _():
            pltpu.sync_copy(tail_hbm, tailbuf)
            pltpu.sync_copy(tailbuf, out_hbm.at[pl.ds(tail_row0, 8)])

    return transpose_kernel(tt, tail)


def kernel(x, y, table):
    B, L = x.shape
    N = B * L
    xt8 = x.T.reshape(8, N // 8)
    yt8 = y.T.reshape(8, N // 8)
    idx8 = jnp.concatenate([xt8, yt8], axis=1).astype(jnp.int32)
    V = table.shape[0]
    tail = jax.lax.slice(table, (V - 64, 0), (V, D)).reshape(8, 128)
    tlin = _sc_transpose(table.T, tail).reshape(V, D)
    g128 = _sc_gather(tlin, idx8)
    seg = jnp.asarray(
        np.kron(np.eye(8, dtype=np.float32), np.ones((D, 1), np.float32)))
    dist8 = _tc_dist(g128, seg)
    return dist8.reshape(L, B).T
